# baseline (speedup 1.0000x reference)
import jax
import jax.numpy as jnp
from jax import lax
from jax.experimental import pallas as pl
from jax.experimental.pallas import tpu as pltpu

_RB = 512


def _body(idx_ref, noise_ref, c_ref, idxv_ref, pairs_hbm, out_ref, rows, sem):
    i = pl.program_id(0)

    def issue(j, _):
        row = idx_ref[i * _RB + j]
        pltpu.make_async_copy(
            pairs_hbm.at[pl.ds(lax.shift_right_logical(row, 1), 1), :],
            rows.at[pl.ds(j, 1), :],
            sem,
        ).start()
        return 0

    lax.fori_loop(0, _RB, issue, 0, unroll=8)

    pltpu.make_async_copy(rows, rows, sem).wait()

    y = lax.dot_general(
        noise_ref[...], c_ref[...],
        dimension_numbers=(((1,), (1,)), ((), ())),
        preferred_element_type=jnp.float32,
    )
    pair = rows[...]
    d = pair.shape[1] // 2
    odd = (idxv_ref[...] & 1) == 1
    gathered = jnp.where(odd, pair[:, d:], pair[:, :d])
    out_ref[...] = y + gathered


def kernel(inds, noise, mns, c):
    B, D = noise.shape
    n = mns.shape[0]
    idx = inds.astype(jnp.int32)
    pairs = mns.reshape(n // 2, 2 * D)
    grid = B // _RB

    return pl.pallas_call(
        _body,
        grid_spec=pltpu.PrefetchScalarGridSpec(
            num_scalar_prefetch=1,
            grid=(grid,),
            in_specs=[
                pl.BlockSpec((_RB, D), lambda i, idx_ref: (i, 0)),
                pl.BlockSpec((D, D), lambda i, idx_ref: (0, 0)),
                pl.BlockSpec((_RB, 1), lambda i, idx_ref: (i, 0)),
                pl.BlockSpec(memory_space=pltpu.MemorySpace.HBM),
            ],
            out_specs=pl.BlockSpec((_RB, D), lambda i, idx_ref: (i, 0)),
            scratch_shapes=[
                pltpu.VMEM((_RB, 2 * D), jnp.float32),
                pltpu.SemaphoreType.DMA,
            ],
        ),
        out_shape=jax.ShapeDtypeStruct((B, D), jnp.float32),
    )(idx, noise, c, idx.reshape(B, 1), pairs)
